# Initial kernel scaffold; baseline (speedup 1.0000x reference)
import jax
import jax.numpy as jnp
from jax.experimental import pallas as pl


def kernel(pred, pred_uvd, target, uvd_gt, train_out):
    raise NotImplementedError("write your pallas kernel here")


# speedup vs baseline: 1.4247x; 1.4247x over previous
import functools

import jax
import jax.numpy as jnp
from jax import lax
from jax.experimental import pallas as pl
from jax.experimental.pallas import tpu as pltpu
from jax.experimental.pallas import tpu_sc as plsc

_ANCHORS = [1.3221, 1.73145, 3.19275, 4.00944, 5.05587, 8.09892,
            9.47112, 4.84053, 11.2364, 10.0071]
_A = 5
_OBJECT_SCALE = 5.0
_SIL_THRESH = 0.6
_L = 16

_F32 = jnp.float32
_I32 = jnp.int32


def _lane_iota():
    return lax.iota(_I32, _L)


def _bcast_lane(v, i):
    idx = jnp.full((_L,), i, _I32)
    dnums = lax.GatherDimensionNumbers(
        offset_dims=(), collapsed_slice_dims=(0,), start_index_map=(0,))
    return lax.gather(v, idx[:, None], dnums, slice_sizes=(1,),
                      mode=lax.GatherScatterMode.PROMISE_IN_BOUNDS)


def _sig(x):
    return 1.0 / (1.0 + jnp.exp(-x))


def _viou(x1, y1, w1, h1, x2, y2, w2, h2):
    mx = jnp.minimum(x1 - w1 * 0.5, x2 - w2 * 0.5)
    bx = jnp.maximum(x1 + w1 * 0.5, x2 + w2 * 0.5)
    my = jnp.minimum(y1 - h1 * 0.5, y2 - h2 * 0.5)
    by = jnp.maximum(y1 + h1 * 0.5, y2 + h2 * 0.5)
    uw = bx - mx
    uh = by - my
    cw = w1 + w2 - uw
    ch = h1 + h2 - uh
    carea = jnp.maximum(cw, 0.0) * jnp.maximum(ch, 0.0)
    uarea = w1 * h1 + w2 * h2 - carea
    return carea / uarea


def _vlog(x):
    bits = lax.bitcast_convert_type(x, _I32)
    e = (bits >> 23) - 127
    mbits = (bits & _I32(0x007FFFFF)) | _I32(0x3F800000)
    m = lax.bitcast_convert_type(mbits, _F32)
    big = m > 1.41421356237
    m = jnp.where(big, m * 0.5, m)
    e = e + jnp.where(big, 1, 0)
    t = m - 1.0
    z = t * t
    p = jnp.full((_L,), 7.0376836292e-2, _F32)
    for c in (-1.1514610310e-1, 1.1676998740e-1, -1.2420140846e-1,
              1.4249322787e-1, -1.6668057665e-1, 2.0000714765e-1,
              -2.4999993993e-1, 3.3333331174e-1):
        p = p * t + _F32(c)
    y = t * z * p - 0.5 * z
    return t + y + e.astype(_F32) * _F32(0.6931471805599453)


def _build_sc_call(B, H, W):
    HW = H * W
    CHW = ((HW + _L - 1) // _L) * _L
    NCHUNK = CHW // _L
    try:
        info = plsc.get_sparse_core_info()
        NC, NS = info.num_cores, info.num_subcores
    except Exception:
        NC, NS = 2, 16
    NW = NC * NS
    BPW = B // NW
    UVD_W = 64 * B // NW
    UVD_CH = UVD_W // _L

    mesh = plsc.VectorSubcoreMesh(core_axis_name="c", subcore_axis_name="s")

    @functools.partial(
        pl.kernel, mesh=mesh,
        out_type=jax.ShapeDtypeStruct((NW, _L), _F32),
        scratch_types=[
            pltpu.VMEM((_A, 5, CHW), _F32),
            pltpu.VMEM((_L,), _F32),
            pltpu.VMEM((UVD_W,), _F32),
            pltpu.VMEM((UVD_W,), _F32),
            pltpu.VMEM((_L,), _F32),
        ],
    )
    def sc_loss(pred_hbm, targ_hbm, pu_hbm, gu_hbm, out_hbm,
                pred_v, targ_v, pu_v, gu_v, out_v):
        wid = lax.axis_index("s") * NC + lax.axis_index("c")
        lane = _lane_iota()
        zero = jnp.zeros((_L,), _F32)
        acc = zero

        for k in range(BPW):
            b = wid * BPW + k
            pltpu.sync_copy(pred_hbm.at[b], pred_v)
            pltpu.sync_copy(targ_hbm.at[b], targ_v)
            tv = targ_v[...]
            gxv = _bcast_lane(tv, 0) * _F32(W)
            gyv = _bcast_lane(tv, 1) * _F32(H)
            gwv = _bcast_lane(tv, 2) * _F32(W)
            ghv = _bcast_lane(tv, 3) * _F32(H)

            bestv = jnp.zeros((_L,), _I32)
            biou = None
            for a in range(_A):
                awa = jnp.full((_L,), _ANCHORS[2 * a], _F32)
                aha = jnp.full((_L,), _ANCHORS[2 * a + 1], _F32)
                au = _viou(zero, zero, awa, aha, zero, zero, gwv, ghv)
                if biou is None:
                    biou = au
                else:
                    upd = au > biou
                    bestv = jnp.where(upd, a, bestv)
                    biou = jnp.where(upd, au, biou)
            awbv = zero
            ahbv = zero
            for a in range(_A):
                hit = bestv == a
                awbv = awbv + jnp.where(hit, _F32(_ANCHORS[2 * a]), 0.0)
                ahbv = ahbv + jnp.where(hit, _F32(_ANCHORS[2 * a + 1]), 0.0)
            lwv = _vlog(gwv / awbv)
            lhv = _vlog(ghv / ahbv)
            giv = gxv.astype(_I32)
            gjv = gyv.astype(_I32)
            dxv = gxv - giv.astype(_F32)
            dyv = gyv - gjv.astype(_F32)
            pselv = gjv * W + giv

            for a in range(_A):
                awa = _F32(_ANCHORS[2 * a])
                aha = _F32(_ANCHORS[2 * a + 1])

                def chunk(c, acc, a=a, awa=awa, aha=aha,
                          gxv=gxv, gyv=gyv, gwv=gwv, ghv=ghv,
                          lwv=lwv, lhv=lhv, dxv=dxv, dyv=dyv,
                          bestv=bestv, pselv=pselv):
                    off = c * _L
                    pos = lane + off
                    wgrid = lax.rem(pos, W).astype(_F32)
                    hgrid = lax.div(pos, W).astype(_F32)
                    validf = jnp.where(pos < HW, _F32(1.0), _F32(0.0))
                    xr = pred_v[a, 0, pl.ds(off, _L)]
                    yr = pred_v[a, 1, pl.ds(off, _L)]
                    twv = pred_v[a, 2, pl.ds(off, _L)]
                    thv = pred_v[a, 3, pl.ds(off, _L)]
                    cr = pred_v[a, 4, pl.ds(off, _L)]
                    sx = _sig(xr)
                    sy = _sig(yr)
                    cf = _sig(cr)
                    bxv = sx + wgrid
                    byv = sy + hgrid
                    bwv = jnp.exp(twv) * awa
                    bhv = jnp.exp(thv) * aha
                    iou = _viou(bxv, byv, bwv, bhv, gxv, gyv, gwv, ghv)
                    m01 = jnp.where(iou > _SIL_THRESH, _F32(0.0), validf)
                    sxc = sx - 0.5
                    syc = sy - 0.5
                    base = (sxc * sxc + syc * syc + twv * twv + thv * thv) * validf
                    selff = (jnp.where(pos == pselv, _F32(1.0), _F32(0.0))
                             * jnp.where(bestv == a, _F32(1.0), _F32(0.0)))
                    ex = sx - dxv
                    ey = sy - dyv
                    ew = twv - lwv
                    eh = thv - lhv
                    ec = cf - iou
                    corr = (ex * ex - sxc * sxc + ey * ey - syc * syc
                            + ew * ew - twv * twv + eh * eh - thv * thv
                            + _OBJECT_SCALE * (ec * ec) - cf * cf * m01)
                    return acc + base + cf * cf * m01 + selff * corr

                acc = lax.fori_loop(0, NCHUNK, chunk, acc)

        pltpu.sync_copy(pu_hbm.at[pl.ds(wid * UVD_W, UVD_W)], pu_v)
        pltpu.sync_copy(gu_hbm.at[pl.ds(wid * UVD_W, UVD_W)], gu_v)

        def uvd_chunk(c, acc):
            off = c * _L
            dv = gu_v[pl.ds(off, _L)] - pu_v[pl.ds(off, _L)]
            return acc + dv * dv

        acc = lax.fori_loop(0, UVD_CH, uvd_chunk, acc)

        out_v[...] = acc * 0.5
        pltpu.sync_copy(out_v, out_hbm.at[wid])

    return sc_loss, CHW


def kernel(pred, pred_uvd, target, uvd_gt, train_out):
    B, H, W = pred.shape[0], pred.shape[2], pred.shape[3]
    HW = H * W
    sc_loss, CHW = _build_sc_call(B, H, W)
    predp = jnp.pad(pred.reshape(B, _A, 5, HW),
                    ((0, 0), (0, 0), (0, 0), (0, CHW - HW)))
    targp = jnp.pad(target, ((0, 0), (0, _L - target.shape[1])))
    pu = jnp.pad(pred_uvd, ((0, 0), (0, 64 - pred_uvd.shape[1]))).reshape(-1)
    gu = jnp.pad(uvd_gt, ((0, 0), (0, 64 - uvd_gt.shape[1]))).reshape(-1)
    partials = sc_loss(predp, targp, pu, gu)
    return jnp.sum(partials)
